# baseline (speedup 1.0000x reference)
import functools

import jax
import jax.numpy as jnp
from jax import lax
from jax.experimental import pallas as pl
from jax.experimental.pallas import tpu as pltpu
from jax.experimental.pallas import tpu_sc as plsc

S = 2048
D = 1024
F = 2816
E = 8
K = 2
M = 256
G = 24
NTOT = G * M
NF = 2
FC = F // NF
EPAD = 128

NC = 2
NS = 16
NW = NC * NS
GPW = NTOT // NW
GCH = 48
GNC = GPW // GCH
CPW = S // NW
CCH = 32


def _router_kernel(x_ref, wr_ref, brp_ref, i1_ref, i2_ref, w1_ref, w2_ref):
    x = x_ref[...]
    logits = jnp.dot(x, wr_ref[...], preferred_element_type=jnp.float32)
    logits = logits + brp_ref[...]
    m1 = jnp.max(logits, axis=-1)
    i1 = jnp.argmax(logits, axis=-1).astype(jnp.int32)
    cols = jax.lax.broadcasted_iota(jnp.int32, logits.shape, 1)
    masked = jnp.where(cols == i1[:, None], -jnp.inf, logits)
    m2 = jnp.max(masked, axis=-1)
    i2 = jnp.argmax(masked, axis=-1).astype(jnp.int32)
    w1 = jax.nn.sigmoid(m1 - m2)
    i1_ref[...] = i1[:, None]
    i2_ref[...] = i2[:, None]
    w1_ref[...] = w1[:, None]
    w2_ref[...] = (1.0 - w1)[:, None]


def _sc_mesh():
    return plsc.VectorSubcoreMesh(
        core_axis_name="c", subcore_axis_name="s",
        num_cores=NC, num_subcores=NS)


@functools.cache
def _make_sc_dispatch():
    @functools.partial(
        pl.kernel,
        mesh=_sc_mesh(),
        out_type=jax.ShapeDtypeStruct((NTOT, D), jnp.float32),
        scratch_types=[
            pltpu.VMEM((CPW, D), jnp.float32),
            pltpu.VMEM((2, CPW), jnp.int32),
            pltpu.SemaphoreType.DMA,
            pltpu.SemaphoreType.DMA,
        ],
    )
    def _sc_dispatch(x_hbm, s1_hbm, s2_hbm, out_hbm, xbuf, idxb, sem1, sem2):
        wid = lax.axis_index("s") * NC + lax.axis_index("c")
        t0 = wid * CPW
        i1h = pltpu.async_copy(s1_hbm.at[pl.ds(t0, CPW)], idxb.at[0], sem1)
        i2h = pltpu.async_copy(s2_hbm.at[pl.ds(t0, CPW)], idxb.at[1], sem2)
        pltpu.sync_copy(x_hbm.at[pl.ds(t0, CPW)], xbuf)
        i1h.wait()
        i2h.wait()
        w1h = pltpu.async_copy(xbuf, out_hbm.at[idxb.at[0]], sem1)
        w2h = pltpu.async_copy(xbuf, out_hbm.at[idxb.at[1]], sem2)
        w1h.wait()
        w2h.wait()
    return _sc_dispatch


@functools.cache
def _make_sc_combine():
    @functools.partial(
        pl.kernel,
        mesh=_sc_mesh(),
        out_type=jax.ShapeDtypeStruct((S, D), jnp.float32),
        scratch_types=[
            pltpu.VMEM((CCH,), jnp.int32),
            pltpu.VMEM((CCH,), jnp.int32),
            pltpu.VMEM((CCH, D), jnp.float32),
            pltpu.VMEM((CCH, D), jnp.float32),
            pltpu.SemaphoreType.DMA,
            pltpu.SemaphoreType.DMA,
        ],
    )
    def _sc_combine(y_hbm, s1_hbm, s2_hbm, out_hbm, i1v, i2v, b1, b2,
                    sem1, sem2):
        wid = lax.axis_index("s") * NC + lax.axis_index("c")
        for ci in range(CPW // CCH):
            base = wid * CPW + ci * CCH
            pltpu.sync_copy(s1_hbm.at[pl.ds(base, CCH)], i1v)
            pltpu.sync_copy(s2_hbm.at[pl.ds(base, CCH)], i2v)
            cp1 = pltpu.async_copy(y_hbm.at[i1v], b1, sem1)
            cp2 = pltpu.async_copy(y_hbm.at[i2v], b2, sem2)
            cp1.wait()
            cp2.wait()

            def rbody(r, _):
                for c in range(D // 16):
                    sl = pl.ds(c * 16, 16)
                    b1[r, sl] = b1[r, sl] + b2[r, sl]
                return 0
            jax.lax.fori_loop(0, CCH, rbody, 0)
            pltpu.sync_copy(b1, out_hbm.at[pl.ds(base, CCH)])
    return _sc_combine


def _ffn_kernel0(eids_ref, nact_ref,
                 xg_ref, w_ref, W1_ref, W3_ref, W2_ref,
                 y_ref):
    g = pl.program_id(0)

    @pl.when(g < nact_ref[0])
    def _active():
        xb = xg_ref[...].astype(jnp.bfloat16)
        h1 = jnp.dot(xb, W1_ref[0].astype(jnp.bfloat16),
                     preferred_element_type=jnp.float32)
        h3 = jnp.dot(xb, W3_ref[0].astype(jnp.bfloat16),
                     preferred_element_type=jnp.float32)
        h = (h1 * jax.nn.sigmoid(h1)) * h3
        y_ref[...] = jnp.dot(h.astype(jnp.bfloat16),
                             W2_ref[0].astype(jnp.bfloat16),
                             preferred_element_type=jnp.float32) * w_ref[0]


def _ffn_kernel1(eids_ref, nact_ref,
                 xg_ref, w_ref, yprev_ref, W1_ref, W3_ref, W2_ref,
                 y_ref):
    g = pl.program_id(0)

    @pl.when(g < nact_ref[0])
    def _active():
        xb = xg_ref[...].astype(jnp.bfloat16)
        h1 = jnp.dot(xb, W1_ref[0].astype(jnp.bfloat16),
                     preferred_element_type=jnp.float32)
        h3 = jnp.dot(xb, W3_ref[0].astype(jnp.bfloat16),
                     preferred_element_type=jnp.float32)
        h = (h1 * jax.nn.sigmoid(h1)) * h3
        y = jnp.dot(h.astype(jnp.bfloat16), W2_ref[0].astype(jnp.bfloat16),
                    preferred_element_type=jnp.float32)
        y_ref[...] = yprev_ref[...] + y * w_ref[0]


def _dispatch(i1, i2, w1, w2):
    e_flat = jnp.concatenate([i1[:, 0], i2[:, 0]])
    t_flat = jnp.concatenate([jnp.arange(S, dtype=jnp.int32)] * 2)
    w_flat = jnp.concatenate([w1[:, 0], w2[:, 0]])
    onehot = (e_flat[:, None] == jnp.arange(E, dtype=jnp.int32)[None, :])
    csum = jnp.cumsum(onehot.astype(jnp.int32), axis=0)
    rank = jnp.sum(jnp.where(onehot, csum, 0), axis=1) - 1
    counts = csum[-1]
    blocks_per = (counts + M - 1) // M
    cumb = jnp.cumsum(blocks_per)
    total_blocks = cumb[-1]
    gidx = jnp.minimum(jnp.arange(G, dtype=jnp.int32), total_blocks - 1)
    eids = jnp.sum(cumb[None, :] <= gidx[:, None], axis=1).astype(jnp.int32)
    pad_start = (jnp.concatenate([jnp.zeros(1, cumb.dtype), cumb[:-1]]) * M)
    slot = (jnp.sum(jnp.where(onehot, pad_start[None, :], 0), axis=1) + rank
            ).astype(jnp.int32)
    wts = jnp.zeros((NTOT,), jnp.float32).at[slot].set(w_flat)
    nact = total_blocks.astype(jnp.int32)[None]
    s1 = slot[:S]
    s2 = slot[S:]
    return eids, nact, wts.reshape(G, M, 1), s1, s2


def _dispatch_rows(xf, s1, s2):
    return _make_sc_dispatch()(xf, s1, s2)


def _combine_rows(y_pad, s1, s2):
    return _make_sc_combine()(y_pad, s1, s2)


def kernel(x, Wr, br, W1, W2, W3):
    xf = x.reshape(S, D)
    wrp = jnp.zeros((D, EPAD), jnp.float32).at[:, :E].set(Wr)
    brp = jnp.full((EPAD,), -jnp.inf, jnp.float32).at[:E].set(br)

    i1, i2, w1, w2 = pl.pallas_call(
        _router_kernel,
        out_shape=[
            jax.ShapeDtypeStruct((S, 1), jnp.int32),
            jax.ShapeDtypeStruct((S, 1), jnp.int32),
            jax.ShapeDtypeStruct((S, 1), jnp.float32),
            jax.ShapeDtypeStruct((S, 1), jnp.float32),
        ],
    )(xf, wrp, brp)

    eids, nact, wts, s1, s2 = _dispatch(i1, i2, w1, w2)

    xg = _dispatch_rows(xf, s1, s2)

    grid_spec0 = pltpu.PrefetchScalarGridSpec(
        num_scalar_prefetch=2,
        grid=(G,),
        in_specs=[
            pl.BlockSpec((M, D), lambda g, eids, nact: (g, 0)),
            pl.BlockSpec((1, M, 1), lambda g, eids, nact: (g, 0, 0)),
            pl.BlockSpec((1, D, FC), lambda g, eids, nact: (eids[g], 0, 0)),
            pl.BlockSpec((1, D, FC), lambda g, eids, nact: (eids[g], 0, 0)),
            pl.BlockSpec((1, FC, D), lambda g, eids, nact: (eids[g], 0, 0)),
        ],
        out_specs=pl.BlockSpec((M, D), lambda g, eids, nact: (g, 0)),
    )

    y0 = pl.pallas_call(
        _ffn_kernel0,
        grid_spec=grid_spec0,
        out_shape=jax.ShapeDtypeStruct((NTOT, D), jnp.float32),
        compiler_params=pltpu.CompilerParams(
            vmem_limit_bytes=60 * 1024 * 1024,
        ),
    )(eids, nact, xg, wts, W1, W3, W2)

    grid_spec1 = pltpu.PrefetchScalarGridSpec(
        num_scalar_prefetch=2,
        grid=(G,),
        in_specs=[
            pl.BlockSpec((M, D), lambda g, eids, nact: (g, 0)),
            pl.BlockSpec((1, M, 1), lambda g, eids, nact: (g, 0, 0)),
            pl.BlockSpec((M, D), lambda g, eids, nact: (g, 0)),
            pl.BlockSpec((1, D, FC), lambda g, eids, nact: (eids[g], 0, 1)),
            pl.BlockSpec((1, D, FC), lambda g, eids, nact: (eids[g], 0, 1)),
            pl.BlockSpec((1, FC, D), lambda g, eids, nact: (eids[g], 1, 0)),
        ],
        out_specs=pl.BlockSpec((M, D), lambda g, eids, nact: (g, 0)),
    )

    y_pad = pl.pallas_call(
        _ffn_kernel1,
        grid_spec=grid_spec1,
        out_shape=jax.ShapeDtypeStruct((NTOT, D), jnp.float32),
        input_output_aliases={4: 0},
        compiler_params=pltpu.CompilerParams(
            vmem_limit_bytes=60 * 1024 * 1024,
        ),
    )(eids, nact, xg, wts, y0, W1, W3, W2)

    out = _combine_rows(y_pad, s1, s2)
    return out.reshape(x.shape)


# speedup vs baseline: 1.0670x; 1.0670x over previous
import functools

import jax
import jax.numpy as jnp
from jax import lax
from jax.experimental import pallas as pl
from jax.experimental.pallas import tpu as pltpu
from jax.experimental.pallas import tpu_sc as plsc

S = 2048
D = 1024
F = 2816
E = 8
K = 2
M = 512
G = 15
NTOT = G * M
NF = 2
FC = F // NF
EPAD = 128

NC = 2
NS = 16
NW = NC * NS
GPW = NTOT // NW
GCH = 48
GNC = GPW // GCH
CPW = S // NW
CCH = 32


def _router_kernel(x_ref, wr_ref, brp_ref, i1_ref, i2_ref, w1_ref, w2_ref):
    x = x_ref[...]
    logits = jnp.dot(x, wr_ref[...], preferred_element_type=jnp.float32)
    logits = logits + brp_ref[...]
    m1 = jnp.max(logits, axis=-1)
    i1 = jnp.argmax(logits, axis=-1).astype(jnp.int32)
    cols = jax.lax.broadcasted_iota(jnp.int32, logits.shape, 1)
    masked = jnp.where(cols == i1[:, None], -jnp.inf, logits)
    m2 = jnp.max(masked, axis=-1)
    i2 = jnp.argmax(masked, axis=-1).astype(jnp.int32)
    w1 = jax.nn.sigmoid(m1 - m2)
    i1_ref[...] = i1[:, None]
    i2_ref[...] = i2[:, None]
    w1_ref[...] = w1[:, None]
    w2_ref[...] = (1.0 - w1)[:, None]


def _sc_mesh():
    return plsc.VectorSubcoreMesh(
        core_axis_name="c", subcore_axis_name="s",
        num_cores=NC, num_subcores=NS)


@functools.cache
def _make_sc_dispatch():
    @functools.partial(
        pl.kernel,
        mesh=_sc_mesh(),
        out_type=jax.ShapeDtypeStruct((NTOT, D), jnp.float32),
        scratch_types=[
            pltpu.VMEM((CPW, D), jnp.float32),
            pltpu.VMEM((2, CPW), jnp.int32),
            pltpu.SemaphoreType.DMA,
            pltpu.SemaphoreType.DMA,
        ],
    )
    def _sc_dispatch(x_hbm, s1_hbm, s2_hbm, out_hbm, xbuf, idxb, sem1, sem2):
        wid = lax.axis_index("s") * NC + lax.axis_index("c")
        t0 = wid * CPW
        i1h = pltpu.async_copy(s1_hbm.at[pl.ds(t0, CPW)], idxb.at[0], sem1)
        i2h = pltpu.async_copy(s2_hbm.at[pl.ds(t0, CPW)], idxb.at[1], sem2)
        pltpu.sync_copy(x_hbm.at[pl.ds(t0, CPW)], xbuf)
        i1h.wait()
        i2h.wait()
        w1h = pltpu.async_copy(xbuf, out_hbm.at[idxb.at[0]], sem1)
        w2h = pltpu.async_copy(xbuf, out_hbm.at[idxb.at[1]], sem2)
        w1h.wait()
        w2h.wait()
    return _sc_dispatch


@functools.cache
def _make_sc_combine():
    @functools.partial(
        pl.kernel,
        mesh=_sc_mesh(),
        out_type=jax.ShapeDtypeStruct((S, D), jnp.float32),
        scratch_types=[
            pltpu.VMEM((CCH,), jnp.int32),
            pltpu.VMEM((CCH,), jnp.int32),
            pltpu.VMEM((CCH, D), jnp.float32),
            pltpu.VMEM((CCH, D), jnp.float32),
            pltpu.SemaphoreType.DMA,
            pltpu.SemaphoreType.DMA,
        ],
    )
    def _sc_combine(y_hbm, s1_hbm, s2_hbm, out_hbm, i1v, i2v, b1, b2,
                    sem1, sem2):
        wid = lax.axis_index("s") * NC + lax.axis_index("c")
        for ci in range(CPW // CCH):
            base = wid * CPW + ci * CCH
            pltpu.sync_copy(s1_hbm.at[pl.ds(base, CCH)], i1v)
            pltpu.sync_copy(s2_hbm.at[pl.ds(base, CCH)], i2v)
            cp1 = pltpu.async_copy(y_hbm.at[i1v], b1, sem1)
            cp2 = pltpu.async_copy(y_hbm.at[i2v], b2, sem2)
            cp1.wait()
            cp2.wait()

            def rbody(r, _):
                for c in range(D // 16):
                    sl = pl.ds(c * 16, 16)
                    b1[r, sl] = b1[r, sl] + b2[r, sl]
                return 0
            jax.lax.fori_loop(0, CCH, rbody, 0)
            pltpu.sync_copy(b1, out_hbm.at[pl.ds(base, CCH)])
    return _sc_combine


def _ffn_kernel0(eids_ref, nact_ref,
                 xg_ref, w_ref, W1_ref, W3_ref, W2_ref,
                 y_ref):
    g = pl.program_id(0)

    @pl.when(g < nact_ref[0])
    def _active():
        xb = xg_ref[...].astype(jnp.bfloat16)
        h1 = jnp.dot(xb, W1_ref[0].astype(jnp.bfloat16),
                     preferred_element_type=jnp.float32)
        h3 = jnp.dot(xb, W3_ref[0].astype(jnp.bfloat16),
                     preferred_element_type=jnp.float32)
        h = (h1 * jax.nn.sigmoid(h1)) * h3
        y_ref[...] = jnp.dot(h.astype(jnp.bfloat16),
                             W2_ref[0].astype(jnp.bfloat16),
                             preferred_element_type=jnp.float32) * w_ref[0]


def _ffn_kernel1(eids_ref, nact_ref,
                 xg_ref, w_ref, yprev_ref, W1_ref, W3_ref, W2_ref,
                 y_ref):
    g = pl.program_id(0)

    @pl.when(g < nact_ref[0])
    def _active():
        xb = xg_ref[...].astype(jnp.bfloat16)
        h1 = jnp.dot(xb, W1_ref[0].astype(jnp.bfloat16),
                     preferred_element_type=jnp.float32)
        h3 = jnp.dot(xb, W3_ref[0].astype(jnp.bfloat16),
                     preferred_element_type=jnp.float32)
        h = (h1 * jax.nn.sigmoid(h1)) * h3
        y = jnp.dot(h.astype(jnp.bfloat16), W2_ref[0].astype(jnp.bfloat16),
                    preferred_element_type=jnp.float32)
        y_ref[...] = yprev_ref[...] + y * w_ref[0]


def _dispatch(i1, i2, w1, w2):
    e_flat = jnp.concatenate([i1[:, 0], i2[:, 0]])
    t_flat = jnp.concatenate([jnp.arange(S, dtype=jnp.int32)] * 2)
    w_flat = jnp.concatenate([w1[:, 0], w2[:, 0]])
    onehot = (e_flat[:, None] == jnp.arange(E, dtype=jnp.int32)[None, :])
    csum = jnp.cumsum(onehot.astype(jnp.int32), axis=0)
    rank = jnp.sum(jnp.where(onehot, csum, 0), axis=1) - 1
    counts = csum[-1]
    blocks_per = (counts + M - 1) // M
    cumb = jnp.cumsum(blocks_per)
    total_blocks = cumb[-1]
    gidx = jnp.minimum(jnp.arange(G, dtype=jnp.int32), total_blocks - 1)
    eids = jnp.sum(cumb[None, :] <= gidx[:, None], axis=1).astype(jnp.int32)
    pad_start = (jnp.concatenate([jnp.zeros(1, cumb.dtype), cumb[:-1]]) * M)
    slot = (jnp.sum(jnp.where(onehot, pad_start[None, :], 0), axis=1) + rank
            ).astype(jnp.int32)
    wts = jnp.zeros((NTOT,), jnp.float32).at[slot].set(w_flat)
    nact = total_blocks.astype(jnp.int32)[None]
    s1 = slot[:S]
    s2 = slot[S:]
    return eids, nact, wts.reshape(G, M, 1), s1, s2


def _dispatch_rows(xf, s1, s2):
    return _make_sc_dispatch()(xf, s1, s2)


def _combine_rows(y_pad, s1, s2):
    return _make_sc_combine()(y_pad, s1, s2)


def kernel(x, Wr, br, W1, W2, W3):
    xf = x.reshape(S, D)
    wrp = jnp.zeros((D, EPAD), jnp.float32).at[:, :E].set(Wr)
    brp = jnp.full((EPAD,), -jnp.inf, jnp.float32).at[:E].set(br)

    i1, i2, w1, w2 = pl.pallas_call(
        _router_kernel,
        out_shape=[
            jax.ShapeDtypeStruct((S, 1), jnp.int32),
            jax.ShapeDtypeStruct((S, 1), jnp.int32),
            jax.ShapeDtypeStruct((S, 1), jnp.float32),
            jax.ShapeDtypeStruct((S, 1), jnp.float32),
        ],
    )(xf, wrp, brp)

    eids, nact, wts, s1, s2 = _dispatch(i1, i2, w1, w2)

    xg = _dispatch_rows(xf, s1, s2)

    grid_spec0 = pltpu.PrefetchScalarGridSpec(
        num_scalar_prefetch=2,
        grid=(G,),
        in_specs=[
            pl.BlockSpec((M, D), lambda g, eids, nact: (g, 0)),
            pl.BlockSpec((1, M, 1), lambda g, eids, nact: (g, 0, 0)),
            pl.BlockSpec((1, D, FC), lambda g, eids, nact: (eids[g], 0, 0)),
            pl.BlockSpec((1, D, FC), lambda g, eids, nact: (eids[g], 0, 0)),
            pl.BlockSpec((1, FC, D), lambda g, eids, nact: (eids[g], 0, 0)),
        ],
        out_specs=pl.BlockSpec((M, D), lambda g, eids, nact: (g, 0)),
    )

    y0 = pl.pallas_call(
        _ffn_kernel0,
        grid_spec=grid_spec0,
        out_shape=jax.ShapeDtypeStruct((NTOT, D), jnp.float32),
        compiler_params=pltpu.CompilerParams(
            vmem_limit_bytes=60 * 1024 * 1024,
        ),
    )(eids, nact, xg, wts, W1, W3, W2)

    grid_spec1 = pltpu.PrefetchScalarGridSpec(
        num_scalar_prefetch=2,
        grid=(G,),
        in_specs=[
            pl.BlockSpec((M, D), lambda g, eids, nact: (g, 0)),
            pl.BlockSpec((1, M, 1), lambda g, eids, nact: (g, 0, 0)),
            pl.BlockSpec((M, D), lambda g, eids, nact: (g, 0)),
            pl.BlockSpec((1, D, FC), lambda g, eids, nact: (eids[g], 0, 1)),
            pl.BlockSpec((1, D, FC), lambda g, eids, nact: (eids[g], 0, 1)),
            pl.BlockSpec((1, FC, D), lambda g, eids, nact: (eids[g], 1, 0)),
        ],
        out_specs=pl.BlockSpec((M, D), lambda g, eids, nact: (g, 0)),
    )

    y_pad = pl.pallas_call(
        _ffn_kernel1,
        grid_spec=grid_spec1,
        out_shape=jax.ShapeDtypeStruct((NTOT, D), jnp.float32),
        input_output_aliases={4: 0},
        compiler_params=pltpu.CompilerParams(
            vmem_limit_bytes=60 * 1024 * 1024,
        ),
    )(eids, nact, xg, wts, y0, W1, W3, W2)

    out = _combine_rows(y_pad, s1, s2)
    return out.reshape(x.shape)
